# baseline (speedup 1.0000x reference)
import functools
import math

import jax
import jax.numpy as jnp
from jax import lax
from jax.experimental import pallas as pl
from jax.experimental.pallas import tpu as pltpu
from jax.experimental.pallas import tpu_sc as plsc

D_MODEL = 64
SCALE = math.sqrt(D_MODEL)
LANES = 16
B, S = 4096, 200
NBJ = B // 128
NSI = S // 8

_info = plsc.get_sparse_core_info()
NC, NS = _info.num_cores, _info.num_subcores


def _emb_body(table_hbm, x4_hbm, out_hbm,
              stage_v, g0, g1, d0, d1, gsem0, gsem1, wsem0, wsem1):
    bj = lax.axis_index("s") * NC + lax.axis_index("c")
    gbuf, dbuf = (g0, g1), (d0, d1)
    gsem, wsem = (gsem0, gsem1), (wsem0, wsem1)

    pltpu.sync_copy(x4_hbm.at[:, bj], stage_v)

    lane = lax.iota(jnp.int32, LANES)
    rowsel = [j * LANES + lane for j in range(128 // LANES)]

    def start_gather(s, b):
        pltpu.async_copy(
            table_hbm.at[stage_v.at[s >> 3, s & 7]], gbuf[b], gsem[b])

    def wait_gather(b):
        pltpu.make_async_copy(
            table_hbm.at[stage_v.at[0, 0]], gbuf[b], gsem[b]).wait()

    def start_write(s, b):
        pltpu.async_copy(dbuf[b], out_hbm.at[s, :, bj], wsem[b])

    def wait_write(b):
        pltpu.make_async_copy(dbuf[b], out_hbm.at[0, :, bj], wsem[b]).wait()

    start_gather(0, 0)
    start_gather(1, 1)

    def do_pair(step, carry):
        for b in (0, 1):
            s = step * 2 + b
            wait_gather(b)

            @pl.when(s >= 2)
            def _():
                wait_write(b)

            def trans_d(dd, c):
                col = jnp.full((LANES,), dd, jnp.int32)
                g = dd >> 3
                roff = (dd & 7) * 128
                for j in range(128 // LANES):
                    vals = plsc.load_gather(gbuf[b], [rowsel[j], col])
                    dbuf[b][g, pl.ds(roff + j * LANES, LANES)] = vals * SCALE
                return c

            lax.fori_loop(0, D_MODEL, trans_d, 0, unroll=4)
            start_write(s, b)

            @pl.when(s + 2 < S)
            def _():
                start_gather(s + 2, b)
        return carry

    lax.fori_loop(0, S // 2, do_pair, 0)
    wait_write(0)
    wait_write(1)


def kernel(x, lut):
    x4 = x.astype(jnp.int32).reshape(NBJ, 128, NSI, 8).transpose(2, 0, 3, 1)

    out4 = pl.kernel(
        _emb_body,
        out_type=jax.ShapeDtypeStruct((S, 8, NBJ, 1024), jnp.float32),
        mesh=plsc.VectorSubcoreMesh(core_axis_name="c", subcore_axis_name="s"),
        compiler_params=pltpu.CompilerParams(
            use_tc_tiling_on_sc=False, needs_layout_passes=False),
        scratch_types=[
            pltpu.VMEM((NSI, 8, 128), jnp.int32),
            pltpu.VMEM((128, D_MODEL), jnp.float32),
            pltpu.VMEM((128, D_MODEL), jnp.float32),
            pltpu.VMEM((8, 1024), jnp.float32),
            pltpu.VMEM((8, 1024), jnp.float32),
            pltpu.SemaphoreType.DMA,
            pltpu.SemaphoreType.DMA,
            pltpu.SemaphoreType.DMA,
            pltpu.SemaphoreType.DMA,
        ],
    )(lut, x4)

    o = (out4.reshape(S, 8, NBJ, 8, 128)
         .transpose(2, 4, 0, 1, 3)
         .reshape(B, S, D_MODEL))
    return o
